# baseline (speedup 1.0000x reference)
import jax
import jax.numpy as jnp
from jax.experimental import pallas as pl
from jax.experimental.pallas import tpu as pltpu

_N = 10000
_D_IN = 128
_D_HID = 64
_D_OUT = 16

_BI = 400
_CK = 2048
_GI = _N // _BI
_GK = pl.cdiv(_N, _CK)
_NPAD = _GK * _CK


def _proj_kernel(x_ref, w1_ref, b1_ref, wg_ref, p_ref):
    h = jnp.dot(x_ref[...], w1_ref[...], preferred_element_type=jnp.float32)
    h = jnp.maximum(h + b1_ref[...], 0.0)
    p_ref[pl.ds(0, _N), :] = jnp.dot(h, wg_ref[...],
                                     preferred_element_type=jnp.float32)
    p_ref[pl.ds(_N, _NPAD - _N), :] = jnp.zeros((_NPAD - _N, _D_OUT),
                                                jnp.float32)


def _agg_kernel(conn_ref, p_ref, bg_ref, out_ref):
    k = pl.program_id(1)
    c = conn_ref[...]

    @pl.when(k == _GK - 1)
    def _masked():
        col = jax.lax.broadcasted_iota(jnp.int32, (_BI, _CK), 1)
        cm = jnp.where(col < _N - k * _CK, c, 0.0)
        out_ref[...] += jnp.dot(cm, p_ref[...],
                                preferred_element_type=jnp.float32)

    @pl.when(k == 0)
    def _first():
        out_ref[...] = jnp.dot(c, p_ref[...],
                               preferred_element_type=jnp.float32) + bg_ref[...]

    @pl.when(jnp.logical_and(k > 0, k < _GK - 1))
    def _mid():
        out_ref[...] += jnp.dot(c, p_ref[...],
                                preferred_element_type=jnp.float32)


def kernel(x, conn, W1, b1, Wg, bg):
    p = pl.pallas_call(
        _proj_kernel,
        in_specs=[
            pl.BlockSpec((_N, _D_IN), lambda: (0, 0)),
            pl.BlockSpec((_D_IN, _D_HID), lambda: (0, 0)),
            pl.BlockSpec((1, _D_HID), lambda: (0, 0)),
            pl.BlockSpec((_D_HID, _D_OUT), lambda: (0, 0)),
        ],
        out_specs=pl.BlockSpec((_NPAD, _D_OUT), lambda: (0, 0)),
        out_shape=jax.ShapeDtypeStruct((_NPAD, _D_OUT), jnp.float32),
    )(x, W1, b1.reshape(1, _D_HID), Wg)

    return pl.pallas_call(
        _agg_kernel,
        grid=(_GI, _GK),
        in_specs=[
            pl.BlockSpec((_BI, _CK), lambda i, k: (i, k)),
            pl.BlockSpec((_CK, _D_OUT), lambda i, k: (k, 0)),
            pl.BlockSpec((1, _D_OUT), lambda i, k: (0, 0)),
        ],
        out_specs=pl.BlockSpec((_BI, _D_OUT), lambda i, k: (i, 0)),
        out_shape=jax.ShapeDtypeStruct((_N, _D_OUT), jnp.float32),
        compiler_params=pltpu.CompilerParams(
            dimension_semantics=("parallel", "arbitrary")),
    )(conn, p, bg.reshape(1, _D_OUT))


# speedup vs baseline: 1.4726x; 1.4726x over previous
import jax
import jax.numpy as jnp
from jax.experimental import pallas as pl
from jax.experimental.pallas import tpu as pltpu

_N = 10000
_D_IN = 128
_D_HID = 64
_D_OUT = 16

_BI = 1000
_CK = 2048
_GI = _N // _BI
_GK = pl.cdiv(_N, _CK)
_NPAD = _GK * _CK


def _fused_kernel(x_ref, conn_ref, w1_ref, b1_ref, wg_ref, bg_ref,
                  out_ref, p_ref):
    i = pl.program_id(0)
    k = pl.program_id(1)

    @pl.when(jnp.logical_and(i == 0, k == 0))
    def _proj():
        h = jnp.dot(x_ref[...], w1_ref[...],
                    preferred_element_type=jnp.float32)
        h = jnp.maximum(h + b1_ref[...], 0.0)
        p = jnp.dot(h, wg_ref[...], preferred_element_type=jnp.float32)
        p_ref[pl.ds(0, _N), :] = p
        p_ref[pl.ds(_N, _NPAD - _N), :] = jnp.zeros(
            (_NPAD - _N, _D_OUT), jnp.float32)

    c = conn_ref[...]

    @pl.when(k == _GK - 1)
    def _masked():
        col = jax.lax.broadcasted_iota(jnp.int32, (_BI, _CK), 1)
        cm = jnp.where(col < _N - k * _CK, c, 0.0)
        out_ref[...] += jnp.dot(cm, p_ref[pl.ds(k * _CK, _CK), :],
                                preferred_element_type=jnp.float32)

    @pl.when(k == 0)
    def _first():
        out_ref[...] = jnp.dot(c, p_ref[pl.ds(0, _CK), :],
                               preferred_element_type=jnp.float32) + bg_ref[...]

    @pl.when(jnp.logical_and(k > 0, k < _GK - 1))
    def _mid():
        out_ref[...] += jnp.dot(c, p_ref[pl.ds(k * _CK, _CK), :],
                                preferred_element_type=jnp.float32)


def kernel(x, conn, W1, b1, Wg, bg):
    return pl.pallas_call(
        _fused_kernel,
        grid=(_GI, _GK),
        in_specs=[
            pl.BlockSpec((_N, _D_IN), lambda i, k: (0, 0)),
            pl.BlockSpec((_BI, _CK), lambda i, k: (i, k)),
            pl.BlockSpec((_D_IN, _D_HID), lambda i, k: (0, 0)),
            pl.BlockSpec((1, _D_HID), lambda i, k: (0, 0)),
            pl.BlockSpec((_D_HID, _D_OUT), lambda i, k: (0, 0)),
            pl.BlockSpec((1, _D_OUT), lambda i, k: (0, 0)),
        ],
        out_specs=pl.BlockSpec((_BI, _D_OUT), lambda i, k: (i, 0)),
        out_shape=jax.ShapeDtypeStruct((_N, _D_OUT), jnp.float32),
        scratch_shapes=[pltpu.MemorySpace.VMEM((_NPAD, _D_OUT), jnp.float32)],
        compiler_params=pltpu.CompilerParams(
            dimension_semantics=("arbitrary", "arbitrary")),
    )(x, conn, W1, b1.reshape(1, _D_HID), Wg, bg.reshape(1, _D_OUT))


# speedup vs baseline: 1.5573x; 1.0576x over previous
import jax
import jax.numpy as jnp
from jax.experimental import pallas as pl
from jax.experimental.pallas import tpu as pltpu

_N = 10000
_D_IN = 128
_D_HID = 64
_D_OUT = 16

_BI = 2000
_CK = 2048
_GI = _N // _BI
_GK = pl.cdiv(_N, _CK)
_NPAD = _GK * _CK


def _fused_kernel(x_ref, conn_ref, w1_ref, b1_ref, wg_ref, bg_ref,
                  out_ref, p_ref):
    i = pl.program_id(0)
    k = pl.program_id(1)

    @pl.when(i == 0)
    def _proj():
        h = jnp.dot(x_ref[...], w1_ref[...],
                    preferred_element_type=jnp.float32)
        h = jnp.maximum(h + b1_ref[...], 0.0)
        pc = jnp.dot(h, wg_ref[...], preferred_element_type=jnp.float32)
        row = jax.lax.broadcasted_iota(jnp.int32, (_CK, _D_OUT), 0)
        p_ref[pl.ds(k * _CK, _CK), :] = jnp.where(row < _N - k * _CK, pc, 0.0)

    c = conn_ref[...]

    @pl.when(k == _GK - 1)
    def _masked():
        col = jax.lax.broadcasted_iota(jnp.int32, (_BI, _CK), 1)
        cm = jnp.where(col < _N - k * _CK, c, 0.0)
        out_ref[...] += jnp.dot(cm, p_ref[pl.ds(k * _CK, _CK), :],
                                preferred_element_type=jnp.float32)

    @pl.when(k == 0)
    def _first():
        out_ref[...] = jnp.dot(c, p_ref[pl.ds(0, _CK), :],
                               preferred_element_type=jnp.float32) + bg_ref[...]

    @pl.when(jnp.logical_and(k > 0, k < _GK - 1))
    def _mid():
        out_ref[...] += jnp.dot(c, p_ref[pl.ds(k * _CK, _CK), :],
                                preferred_element_type=jnp.float32)


def kernel(x, conn, W1, b1, Wg, bg):
    return pl.pallas_call(
        _fused_kernel,
        grid=(_GI, _GK),
        in_specs=[
            pl.BlockSpec((_CK, _D_IN), lambda i, k: (k, 0)),
            pl.BlockSpec((_BI, _CK), lambda i, k: (i, k)),
            pl.BlockSpec((_D_IN, _D_HID), lambda i, k: (0, 0)),
            pl.BlockSpec((1, _D_HID), lambda i, k: (0, 0)),
            pl.BlockSpec((_D_HID, _D_OUT), lambda i, k: (0, 0)),
            pl.BlockSpec((1, _D_OUT), lambda i, k: (0, 0)),
        ],
        out_specs=pl.BlockSpec((_BI, _D_OUT), lambda i, k: (i, 0)),
        out_shape=jax.ShapeDtypeStruct((_N, _D_OUT), jnp.float32),
        scratch_shapes=[pltpu.MemorySpace.VMEM((_NPAD, _D_OUT), jnp.float32)],
        compiler_params=pltpu.CompilerParams(
            dimension_semantics=("arbitrary", "arbitrary")),
    )(x, conn, W1, b1.reshape(1, _D_HID), Wg, bg.reshape(1, _D_OUT))


# speedup vs baseline: 1.6365x; 1.0508x over previous
import jax
import jax.numpy as jnp
from jax.experimental import pallas as pl
from jax.experimental.pallas import tpu as pltpu

_N = 10000
_D_IN = 128
_D_HID = 64
_D_OUT = 16

_BI = 400
_GI = _N // _BI


def _fused_kernel(x_ref, conn_ref, w1_ref, b1_ref, wg_ref, bg_ref,
                  out_ref, p_ref):
    i = pl.program_id(0)

    @pl.when(i == 0)
    def _proj():
        h = jnp.dot(x_ref[...], w1_ref[...],
                    preferred_element_type=jnp.float32)
        h = jnp.maximum(h + b1_ref[...], 0.0)
        p_ref[...] = jnp.dot(h, wg_ref[...],
                             preferred_element_type=jnp.float32)

    out_ref[...] = jnp.dot(conn_ref[...], p_ref[...],
                           preferred_element_type=jnp.float32) + bg_ref[...]


def kernel(x, conn, W1, b1, Wg, bg):
    return pl.pallas_call(
        _fused_kernel,
        grid=(_GI,),
        in_specs=[
            pl.BlockSpec((_N, _D_IN), lambda i: (0, 0)),
            pl.BlockSpec((_BI, _N), lambda i: (i, 0)),
            pl.BlockSpec((_D_IN, _D_HID), lambda i: (0, 0)),
            pl.BlockSpec((1, _D_HID), lambda i: (0, 0)),
            pl.BlockSpec((_D_HID, _D_OUT), lambda i: (0, 0)),
            pl.BlockSpec((1, _D_OUT), lambda i: (0, 0)),
        ],
        out_specs=pl.BlockSpec((_BI, _D_OUT), lambda i: (i, 0)),
        out_shape=jax.ShapeDtypeStruct((_N, _D_OUT), jnp.float32),
        scratch_shapes=[pltpu.MemorySpace.VMEM((_N, _D_OUT), jnp.float32)],
        compiler_params=pltpu.CompilerParams(
            dimension_semantics=("arbitrary",)),
    )(x, conn, W1, b1.reshape(1, _D_HID), Wg, bg.reshape(1, _D_OUT))
